# baseline (speedup 1.0000x reference)
import functools

import jax
import jax.numpy as jnp
from jax import lax
from jax.experimental import pallas as pl
from jax.experimental.pallas import tpu as pltpu
from jax.experimental.pallas import tpu_sc as plsc

_NC = 2
_NS = 16
_NW = _NC * _NS
_L = 16

_CHUNK_ROWS = 16
_NSPLIT = 2


@functools.cache
def _make_sc_add(B, S, D):
    seq_w = S // _NW
    C = _CHUNK_ROWS
    nch = seq_w // C
    nj = D // _L
    npair = B // 2

    mesh = plsc.VectorSubcoreMesh(
        core_axis_name="c", subcore_axis_name="s",
        num_cores=_NC, num_subcores=_NS)

    def body(x_hbm, t_hbm, o_hbm,
             xb0, xb1, xb2, xb3, ob0, ob1, ob2, ob3, tb0, tb1,
             slx0, slx1, slx2, slx3, sst0, sst1, sst2, sst3, slt0, slt1):
        wid = lax.axis_index("s") * _NC + lax.axis_index("c")
        r0 = wid * seq_w

        xbufs = (xb0, xb1, xb2, xb3)
        obufs = (ob0, ob1, ob2, ob3)
        tbufs = (tb0, tb1)
        slx = (slx0, slx1, slx2, slx3)
        sst = (sst0, sst1, sst2, sst3)
        slt = (slt0, slt1)

        def start_load_t(c, k):
            pltpu.async_copy(t_hbm.at[pl.ds(r0 + c * C, C), :], tbufs[k], slt[k])

        def wait_load_t(k):
            pltpu.make_async_copy(t_hbm.at[pl.ds(0, C), :], tbufs[k], slt[k]).wait()

        H = C // _NSPLIT

        def start_load_x(c, b, k):
            r = r0 + c * C
            for q in range(_NSPLIT):
                pltpu.async_copy(
                    x_hbm.at[b, pl.ds(r + q * H, H), :],
                    xbufs[k].at[pl.ds(q * H, H), :], slx[k])

        def wait_load_x(k):
            pltpu.make_async_copy(x_hbm.at[0, pl.ds(0, C), :], xbufs[k], slx[k]).wait()

        def start_store(ob_, c, b, k):
            r = r0 + c * C
            for q in range(_NSPLIT):
                pltpu.async_copy(
                    ob_.at[pl.ds(q * H, H), :],
                    o_hbm.at[b, pl.ds(r + q * H, H), :], sst[k])

        def wait_store(k):
            pltpu.make_async_copy(obufs[k], o_hbm.at[0, pl.ds(0, C), :], sst[k]).wait()

        start_load_t(0, 0)
        start_load_t(1, 1)
        for b in range(B):
            start_load_x(0, b, b)

        @pl.loop(0, nch, step=2)
        def _chunks(c):
            for tk in (0, 1):
                cc = c + tk
                wait_load_t(tk)
                for p in range(npair):
                    b0, b1 = 2 * p, 2 * p + 1
                    k0, k1 = 2 * p, 2 * p + 1

                    @pl.when(cc >= 1)
                    def _():
                        wait_store(k0)
                        wait_store(k1)

                    wait_load_x(k0)
                    wait_load_x(k1)
                    xa, xc = xbufs[k0], xbufs[k1]
                    oa, oc = obufs[k0], obufs[k1]
                    tb = tbufs[tk]

                    r_ = r0 + cc * C
                    for h in range(_NSPLIT):
                        @plsc.parallel_loop(h * H, (h + 1) * H, step=1, unroll=2)
                        def _add(r):
                            for j in range(nj):
                                sl = pl.ds(j * _L, _L)
                                vt = tb[r, sl]
                                oa[r, sl] = xa[r, sl] + vt
                                oc[r, sl] = xc[r, sl] + vt

                        rows = pl.ds(h * H, H)
                        hrows = pl.ds(r_ + h * H, H)
                        pltpu.async_copy(oa.at[rows, :], o_hbm.at[b0, hrows, :], sst[k0])
                        pltpu.async_copy(oc.at[rows, :], o_hbm.at[b1, hrows, :], sst[k1])

                    @pl.when(cc + 1 < nch)
                    def _():
                        start_load_x(cc + 1, b0, k0)
                        start_load_x(cc + 1, b1, k1)

                @pl.when(cc + 2 < nch)
                def _():
                    start_load_t(cc + 2, tk)

        for k in range(2 * npair):
            wait_store(k)

    f32 = jnp.float32
    return pl.kernel(
        body,
        out_type=jax.ShapeDtypeStruct((B, S, D), f32),
        mesh=mesh,
        scratch_types=(
            [pltpu.VMEM((C, D), f32)] * 10
            + [pltpu.SemaphoreType.DMA] * 10
        ),
        compiler_params=pltpu.CompilerParams(use_tc_tiling_on_sc=True),
    )


def kernel(x, pos_emb_table):
    B, S, D = x.shape
    return _make_sc_add(B, S, D)(x, pos_emb_table)


# speedup vs baseline: 1.1860x; 1.1860x over previous
import functools

import jax
import jax.numpy as jnp
from jax import lax
from jax.experimental import pallas as pl
from jax.experimental.pallas import tpu as pltpu
from jax.experimental.pallas import tpu_sc as plsc

_NC = 2
_NS = 16
_NW = _NC * _NS
_L = 16

_CHUNK_ROWS = 16
_NSPLIT = 2


@functools.cache
def _make_sc_add(B, S, D):
    seq_w = S // _NW
    C = _CHUNK_ROWS
    nch = seq_w // C
    nj = D // _L
    npair = B // 2

    mesh = plsc.VectorSubcoreMesh(
        core_axis_name="c", subcore_axis_name="s",
        num_cores=_NC, num_subcores=_NS)

    def body(x_hbm, t_hbm, o_hbm,
             xb0, xb1, xb2, xb3, ob0, ob1, ob2, ob3, tb0, tb1,
             slx0, slx1, slx2, slx3, sst0, sst1, sst2, sst3, slt0, slt1):
        wid = lax.axis_index("s") * _NC + lax.axis_index("c")
        r0 = wid * seq_w

        xbufs = (xb0, xb1, xb2, xb3)
        obufs = (ob0, ob1, ob2, ob3)
        tbufs = (tb0, tb1)
        slx = (slx0, slx1, slx2, slx3)
        sst = (sst0, sst1, sst2, sst3)
        slt = (slt0, slt1)

        def start_load_t(c, k):
            pltpu.async_copy(t_hbm.at[pl.ds(r0 + c * C, C), :], tbufs[k], slt[k])

        def wait_load_t(k):
            pltpu.make_async_copy(t_hbm.at[pl.ds(0, C), :], tbufs[k], slt[k]).wait()

        H = C // _NSPLIT

        def start_load_x(c, b, k):
            r = r0 + c * C
            for q in range(_NSPLIT):
                pltpu.async_copy(
                    x_hbm.at[b, pl.ds(r + q * H, H), :],
                    xbufs[k].at[pl.ds(q * H, H), :], slx[k])

        def wait_load_x(k):
            pltpu.make_async_copy(x_hbm.at[0, pl.ds(0, C), :], xbufs[k], slx[k]).wait()

        def start_store(ob_, c, b, k):
            r = r0 + c * C
            for q in range(_NSPLIT):
                pltpu.async_copy(
                    ob_.at[pl.ds(q * H, H), :],
                    o_hbm.at[b, pl.ds(r + q * H, H), :], sst[k])

        def wait_store(k):
            pltpu.make_async_copy(obufs[k], o_hbm.at[0, pl.ds(0, C), :], sst[k]).wait()

        start_load_t(0, 0)
        start_load_t(1, 1)
        for b in range(B):
            start_load_x(0, b, b)

        @pl.loop(0, nch, step=2)
        def _chunks(c):
            for tk in (0, 1):
                cc = c + tk
                wait_load_t(tk)
                for p in range(npair):
                    b0, b1 = 2 * p, 2 * p + 1
                    k0, k1 = 2 * p, 2 * p + 1

                    @pl.when(cc >= 1)
                    def _():
                        wait_store(k0)
                        wait_store(k1)

                    wait_load_x(k0)
                    wait_load_x(k1)
                    xa, xc = xbufs[k0], xbufs[k1]
                    oa, oc = obufs[k0], obufs[k1]
                    tb = tbufs[tk]

                    @plsc.parallel_loop(0, C, step=1, unroll=2)
                    def _add(r):
                        for j in range(nj):
                            sl = pl.ds(j * _L, _L)
                            vt = tb[r, sl]
                            oa[r, sl] = xa[r, sl] + vt
                            oc[r, sl] = xc[r, sl] + vt

                    start_store(oa, cc, b0, k0)
                    start_store(oc, cc, b1, k1)

                    @pl.when(cc + 1 < nch)
                    def _():
                        start_load_x(cc + 1, b0, k0)
                        start_load_x(cc + 1, b1, k1)

                @pl.when(cc + 2 < nch)
                def _():
                    start_load_t(cc + 2, tk)

        for k in range(2 * npair):
            wait_store(k)

    f32 = jnp.float32
    return pl.kernel(
        body,
        out_type=jax.ShapeDtypeStruct((B, S, D), f32),
        mesh=mesh,
        scratch_types=(
            [pltpu.VMEM((C, D), f32)] * 10
            + [pltpu.SemaphoreType.DMA] * 10
        ),
        compiler_params=pltpu.CompilerParams(use_tc_tiling_on_sc=True),
    )


def kernel(x, pos_emb_table):
    B, S, D = x.shape
    return _make_sc_add(B, S, D)(x, pos_emb_table)
